# baseline (speedup 1.0000x reference)
import functools

import jax
import jax.numpy as jnp
from jax import lax
from jax.experimental import pallas as pl
from jax.experimental.pallas import tpu as pltpu
from jax.experimental.pallas import tpu_sc as plsc

N = 10000
NP = 10240
D_IN = 128
D_HID = 128
D_LAT = 64
E = 320000

NC = 2
NS = 16
LANES = 16
GRP = 128
G = 80
E_PAD = NC * NS * G * GRP
STRIPE = NP // NS

_MESH = plsc.VectorSubcoreMesh(
    core_axis_name="c", subcore_axis_name="s", num_cores=NC, num_subcores=NS
)


def _slab_copy(ei3, tail3, row, wid, p, rpt, pr, dstref):
    @pl.when(wid < NC * NS - 1)
    def _():
        pltpu.sync_copy(ei3.at[row, pl.ds(wid * rpt + p * pr, pr), :], dstref)

    @pl.when(wid == NC * NS - 1)
    def _():
        pltpu.sync_copy(tail3.at[row, pl.ds(p * pr, pr), :], dstref)


def _deg_body(ei3_hbm, tail3_hbm, out_hbm, shared, idx_v, ones_v, zero_v):
    c = lax.axis_index("c")
    s = lax.axis_index("s")
    wid = c * NS + s

    @pl.loop(0, STRIPE // LANES)
    def _(k):
        zero_v[pl.ds(k * LANES, LANES)] = jnp.zeros((LANES,), jnp.float32)

    @pl.loop(0, GRP // LANES)
    def _(k):
        ones_v[pl.ds(k * LANES, LANES)] = jnp.ones((LANES,), jnp.float32)

    pltpu.sync_copy(zero_v, shared.at[pl.ds(s * STRIPE, STRIPE)])
    _slab_copy(ei3_hbm, tail3_hbm, 1, wid, 0, G, G, idx_v)
    plsc.subcore_barrier()

    @pl.loop(0, G)
    def _(j):
        pltpu.sync_copy(ones_v, shared.at[idx_v.at[j]], add=True)

    plsc.subcore_barrier()
    pltpu.sync_copy(
        shared.at[pl.ds(s * STRIPE, STRIPE)],
        out_hbm.at[c, pl.ds(s * STRIPE, STRIPE)],
    )


_deg_kernel = functools.partial(
    pl.kernel,
    out_type=jax.ShapeDtypeStruct((NC, NP), jnp.float32),
    mesh=_MESH,
    scratch_types=[
        pltpu.VMEM_SHARED((NP,), jnp.float32),
        pltpu.VMEM((G, GRP), jnp.int32),
        pltpu.VMEM((GRP,), jnp.float32),
        pltpu.VMEM((STRIPE,), jnp.float32),
    ],
)(_deg_body)


def _make_agg(D, tc_tiling, phases, nbuf, grp, gtot, lookahead=2):
    assert NC * NS * gtot * grp == E_PAD
    GP = gtot // phases
    L = lookahead
    assert GP % nbuf == 0 and 2 <= L < nbuf

    RPT = (E_PAD // (NC * NS)) // grp
    PR = RPT // phases

    def body(g_hbm, ei3_hbm, tail3_hbm, out_hbm, shared, srcv, dstv, *rest):
        bufs = rest[:nbuf]
        gsems = rest[nbuf:2 * nbuf]
        ssems = rest[2 * nbuf:3 * nbuf]
        c = lax.axis_index("c")
        s = lax.axis_index("s")
        wid = c * NS + s

        def gather(j, u):
            pltpu.async_copy(g_hbm.at[srcv.at[j]], bufs[u], gsems[u])

        def gather_wait(u):
            pltpu.make_async_copy(g_hbm.at[srcv.at[0]], bufs[u], gsems[u]).wait()

        def scat(j, u):
            pltpu.async_copy(bufs[u], shared.at[dstv.at[j]], ssems[u], add=True)

        def scat_wait(u):
            pltpu.make_async_copy(bufs[u], shared.at[dstv.at[0]], ssems[u]).wait()

        pltpu.sync_copy(
            g_hbm.at[pl.ds(s * STRIPE, STRIPE), :],
            shared.at[pl.ds(s * STRIPE, STRIPE), :],
        )
        plsc.subcore_barrier()

        @pl.loop(0, phases)
        def _(p):
            _slab_copy(ei3_hbm, tail3_hbm, 0, wid, p, RPT, PR, srcv)
            _slab_copy(ei3_hbm, tail3_hbm, 1, wid, p, RPT, PR, dstv)
            for l in range(L):
                gather(l, l)

            @pl.loop(0, GP // nbuf)
            def _(k):
                jb = nbuf * k
                for u in range(nbuf):
                    j = jb + u
                    gather_wait(u)
                    scat(j, u)
                    nu = (u + L) % nbuf
                    nj = j + L

                    @pl.when(nj < GP)
                    def _():
                        @pl.when(nj - nbuf >= 0)
                        def _():
                            scat_wait(nu)

                        gather(nj, nu)

            for u in range(nbuf):
                scat_wait(u)

        plsc.subcore_barrier()
        pltpu.sync_copy(
            shared.at[pl.ds(s * STRIPE, STRIPE), :],
            out_hbm.at[c, pl.ds(s * STRIPE, STRIPE), :],
        )

    return functools.partial(
        pl.kernel,
        out_type=jax.ShapeDtypeStruct((NC, NP, D), jnp.float32),
        mesh=_MESH,
        scratch_types=[
            pltpu.VMEM_SHARED((NP, D), jnp.float32),
            pltpu.VMEM((GP, grp), jnp.int32),
            pltpu.VMEM((GP, grp), jnp.int32),
        ]
        + [pltpu.VMEM((grp, D), jnp.float32)] * nbuf
        + [pltpu.SemaphoreType.DMA] * (2 * nbuf),
        compiler_params=pltpu.CompilerParams(use_tc_tiling_on_sc=tc_tiling),
    )(body)


_agg_hid = _make_agg(D_HID, False, 2, 8, 32, 320, lookahead=7)
_agg_lat = _make_agg(D_LAT, False, 1, 8, 128, 80, lookahead=6)


_R = 1024


def _k1a_body(x_ref, w_ref, h_ref):
    h_ref[...] = jnp.dot(x_ref[...], w_ref[...], preferred_element_type=jnp.float32)


def _k1b_body(h_ref, deg_ref, g_ref, dinv_ref):
    deg = deg_ref[0] + deg_ref[1] + 1.0
    dinv = lax.rsqrt(deg)[:, None]
    g_ref[...] = h_ref[...] * dinv
    dinv_ref[...] = dinv


def _k3_body(acc_ref, g1_ref, dinv_ref, b1_ref, w2_ref, g2_ref):
    dinv = dinv_ref[...]
    agg = acc_ref[0] + acc_ref[1] - g1_ref[...]
    h = jnp.maximum(agg * dinv + b1_ref[...], 0.0)
    g2_ref[...] = (
        jnp.dot(h, w2_ref[...], preferred_element_type=jnp.float32) * dinv
    )


def _k5_body(acc_ref, g2_ref, dinv_ref, b2_ref, out_ref):
    dinv = dinv_ref[...]
    agg = acc_ref[0] + acc_ref[1] - g2_ref[...]
    out_ref[...] = agg * dinv + b2_ref[...]


def _tc_matmul(x, W1):
    return pl.pallas_call(
        _k1a_body,
        grid=(N // _R5,),
        in_specs=[
            pl.BlockSpec((_R5, D_IN), lambda i: (i, 0)),
            pl.BlockSpec((D_IN, D_HID), lambda i: (0, 0)),
        ],
        out_specs=pl.BlockSpec((_R5, D_HID), lambda i: (i, 0)),
        out_shape=jax.ShapeDtypeStruct((NP, D_HID), jnp.float32),
    )(x, W1)


def _tc_scale(h1, degs):
    return pl.pallas_call(
        _k1b_body,
        grid=(NP // _R,),
        in_specs=[
            pl.BlockSpec((_R, D_HID), lambda i: (i, 0)),
            pl.BlockSpec((NC, _R), lambda i: (0, i)),
        ],
        out_specs=[
            pl.BlockSpec((_R, D_HID), lambda i: (i, 0)),
            pl.BlockSpec((_R, 1), lambda i: (i, 0)),
        ],
        out_shape=[
            jax.ShapeDtypeStruct((NP, D_HID), jnp.float32),
            jax.ShapeDtypeStruct((NP, 1), jnp.float32),
        ],
    )(h1, degs)


def _tc_mid(acc1, g1, dinv, b1r, W2):
    return pl.pallas_call(
        _k3_body,
        grid=(NP // _R,),
        in_specs=[
            pl.BlockSpec((NC, _R, D_HID), lambda i: (0, i, 0)),
            pl.BlockSpec((_R, D_HID), lambda i: (i, 0)),
            pl.BlockSpec((_R, 1), lambda i: (i, 0)),
            pl.BlockSpec((1, D_HID), lambda i: (0, 0)),
            pl.BlockSpec((D_HID, D_LAT), lambda i: (0, 0)),
        ],
        out_specs=pl.BlockSpec((_R, D_LAT), lambda i: (i, 0)),
        out_shape=jax.ShapeDtypeStruct((NP, D_LAT), jnp.float32),
    )(acc1, g1, dinv, b1r, W2)


_R5 = 1000


def _tc_final(acc2, g2, dinv, b2r):
    return pl.pallas_call(
        _k5_body,
        grid=(N // _R5,),
        in_specs=[
            pl.BlockSpec((NC, _R5, D_LAT), lambda i: (0, i, 0)),
            pl.BlockSpec((_R5, D_LAT), lambda i: (i, 0)),
            pl.BlockSpec((_R5, 1), lambda i: (i, 0)),
            pl.BlockSpec((1, D_LAT), lambda i: (0, 0)),
        ],
        out_specs=pl.BlockSpec((_R5, D_LAT), lambda i: (i, 0)),
        out_shape=jax.ShapeDtypeStruct((N, D_LAT), jnp.float32),
    )(acc2, g2, dinv, b2r)


def kernel(x, edge_index, W1, b1, W2, b2):
    ei = edge_index.astype(jnp.int32)
    epw = E_PAD // (NC * NS)
    pad_idx = N + (jnp.arange(E_PAD - E, dtype=jnp.int32) % (NP - N))
    tail = jnp.concatenate(
        [ei[:, (NC * NS - 1) * epw:], jnp.stack([pad_idx, pad_idx])], axis=1
    )
    ei3a = ei.reshape(2, E // 32, 32)
    ei3b = ei.reshape(2, E // 128, 128)
    tail3a = tail.reshape(2, epw // 32, 32)
    tail3b = tail.reshape(2, epw // 128, 128)

    h1 = _tc_matmul(x, W1)
    degs = _deg_kernel(ei3b, tail3b)
    g1, dinv = _tc_scale(h1, degs)
    acc1 = _agg_hid(g1, ei3a, tail3a)
    g2 = _tc_mid(acc1, g1, dinv, b1.reshape(1, D_HID), W2)
    acc2 = _agg_lat(g2, ei3b, tail3b)
    return _tc_final(acc2, g2, dinv, b2.reshape(1, D_LAT))


# speedup vs baseline: 1.0039x; 1.0039x over previous
import functools

import jax
import jax.numpy as jnp
from jax import lax
from jax.experimental import pallas as pl
from jax.experimental.pallas import tpu as pltpu
from jax.experimental.pallas import tpu_sc as plsc

N = 10000
NP = 10240
D_IN = 128
D_HID = 128
D_LAT = 64
E = 320000

NC = 2
NS = 16
LANES = 16
GRP = 128
G = 80
E_PAD = NC * NS * G * GRP
STRIPE = NP // NS

_MESH = plsc.VectorSubcoreMesh(
    core_axis_name="c", subcore_axis_name="s", num_cores=NC, num_subcores=NS
)


def _slab_copy(ei3, tail3, row, wid, p, rpt, pr, dstref):
    @pl.when(wid < NC * NS - 1)
    def _():
        pltpu.sync_copy(ei3.at[row, pl.ds(wid * rpt + p * pr, pr), :], dstref)

    @pl.when(wid == NC * NS - 1)
    def _():
        pltpu.sync_copy(tail3.at[row, pl.ds(p * pr, pr), :], dstref)


def _deg_body(ei3_hbm, tail3_hbm, out_hbm, shared, idx_v, ones_v, zero_v):
    c = lax.axis_index("c")
    s = lax.axis_index("s")
    wid = c * NS + s

    @pl.loop(0, STRIPE // LANES)
    def _(k):
        zero_v[pl.ds(k * LANES, LANES)] = jnp.zeros((LANES,), jnp.float32)

    @pl.loop(0, GRP // LANES)
    def _(k):
        ones_v[pl.ds(k * LANES, LANES)] = jnp.ones((LANES,), jnp.float32)

    pltpu.sync_copy(zero_v, shared.at[pl.ds(s * STRIPE, STRIPE)])
    _slab_copy(ei3_hbm, tail3_hbm, 1, wid, 0, G, G, idx_v)
    plsc.subcore_barrier()

    @pl.loop(0, G)
    def _(j):
        pltpu.sync_copy(ones_v, shared.at[idx_v.at[j]], add=True)

    plsc.subcore_barrier()
    pltpu.sync_copy(
        shared.at[pl.ds(s * STRIPE, STRIPE)],
        out_hbm.at[c, pl.ds(s * STRIPE, STRIPE)],
    )


_deg_kernel = functools.partial(
    pl.kernel,
    out_type=jax.ShapeDtypeStruct((NC, NP), jnp.float32),
    mesh=_MESH,
    scratch_types=[
        pltpu.VMEM_SHARED((NP,), jnp.float32),
        pltpu.VMEM((G, GRP), jnp.int32),
        pltpu.VMEM((GRP,), jnp.float32),
        pltpu.VMEM((STRIPE,), jnp.float32),
    ],
)(_deg_body)


def _make_agg(D, tc_tiling, phases, nbuf, grp, gtot, lookahead=2):
    assert NC * NS * gtot * grp == E_PAD
    GP = gtot // phases
    L = lookahead
    assert GP % nbuf == 0 and 2 <= L < nbuf

    RPT = (E_PAD // (NC * NS)) // grp
    PR = RPT // phases

    def body(g_hbm, ei3_hbm, tail3_hbm, out_hbm, shared, srcv, dstv, *rest):
        bufs = rest[:nbuf]
        gsems = rest[nbuf:2 * nbuf]
        ssems = rest[2 * nbuf:3 * nbuf]
        c = lax.axis_index("c")
        s = lax.axis_index("s")
        wid = c * NS + s

        def gather(j, u):
            pltpu.async_copy(g_hbm.at[srcv.at[j]], bufs[u], gsems[u])

        def gather_wait(u):
            pltpu.make_async_copy(g_hbm.at[srcv.at[0]], bufs[u], gsems[u]).wait()

        def scat(j, u):
            pltpu.async_copy(bufs[u], shared.at[dstv.at[j]], ssems[u], add=True)

        def scat_wait(u):
            pltpu.make_async_copy(bufs[u], shared.at[dstv.at[0]], ssems[u]).wait()

        pltpu.sync_copy(
            g_hbm.at[pl.ds(s * STRIPE, STRIPE), :],
            shared.at[pl.ds(s * STRIPE, STRIPE), :],
        )
        plsc.subcore_barrier()

        @pl.loop(0, phases)
        def _(p):
            _slab_copy(ei3_hbm, tail3_hbm, 0, wid, p, RPT, PR, srcv)
            _slab_copy(ei3_hbm, tail3_hbm, 1, wid, p, RPT, PR, dstv)
            for l in range(L):
                gather(l, l)

            @pl.loop(0, GP // nbuf)
            def _(k):
                jb = nbuf * k
                for u in range(nbuf):
                    j = jb + u
                    gather_wait(u)
                    scat(j, u)
                    nu = (u + L) % nbuf
                    nj = j + L

                    @pl.when(nj < GP)
                    def _():
                        @pl.when(nj - nbuf >= 0)
                        def _():
                            scat_wait(nu)

                        gather(nj, nu)

            for u in range(nbuf):
                scat_wait(u)

        plsc.subcore_barrier()
        pltpu.sync_copy(
            shared.at[pl.ds(s * STRIPE, STRIPE), :],
            out_hbm.at[c, pl.ds(s * STRIPE, STRIPE), :],
        )

    return functools.partial(
        pl.kernel,
        out_type=jax.ShapeDtypeStruct((NC, NP, D), jnp.float32),
        mesh=_MESH,
        scratch_types=[
            pltpu.VMEM_SHARED((NP, D), jnp.float32),
            pltpu.VMEM((GP, grp), jnp.int32),
            pltpu.VMEM((GP, grp), jnp.int32),
        ]
        + [pltpu.VMEM((grp, D), jnp.float32)] * nbuf
        + [pltpu.SemaphoreType.DMA] * (2 * nbuf),
        compiler_params=pltpu.CompilerParams(use_tc_tiling_on_sc=tc_tiling),
    )(body)


_agg_hid = _make_agg(D_HID, False, 2, 8, 32, 320, lookahead=6)
_agg_lat = _make_agg(D_LAT, False, 1, 8, 128, 80, lookahead=6)


_R = 1024


def _k1a_body(x_ref, w_ref, h_ref):
    h_ref[...] = jnp.dot(x_ref[...], w_ref[...], preferred_element_type=jnp.float32)


def _k1b_body(h_ref, deg_ref, g_ref, dinv_ref):
    deg = deg_ref[0] + deg_ref[1] + 1.0
    dinv = lax.rsqrt(deg)[:, None]
    g_ref[...] = h_ref[...] * dinv
    dinv_ref[...] = dinv


def _k3_body(acc_ref, g1_ref, dinv_ref, b1_ref, w2_ref, g2_ref):
    dinv = dinv_ref[...]
    agg = acc_ref[0] + acc_ref[1] - g1_ref[...]
    h = jnp.maximum(agg * dinv + b1_ref[...], 0.0)
    g2_ref[...] = (
        jnp.dot(h, w2_ref[...], preferred_element_type=jnp.float32) * dinv
    )


def _k5_body(acc_ref, g2_ref, dinv_ref, b2_ref, out_ref):
    dinv = dinv_ref[...]
    agg = acc_ref[0] + acc_ref[1] - g2_ref[...]
    out_ref[...] = agg * dinv + b2_ref[...]


def _tc_matmul(x, W1):
    return pl.pallas_call(
        _k1a_body,
        grid=(N // _R5,),
        in_specs=[
            pl.BlockSpec((_R5, D_IN), lambda i: (i, 0)),
            pl.BlockSpec((D_IN, D_HID), lambda i: (0, 0)),
        ],
        out_specs=pl.BlockSpec((_R5, D_HID), lambda i: (i, 0)),
        out_shape=jax.ShapeDtypeStruct((NP, D_HID), jnp.float32),
    )(x, W1)


def _tc_scale(h1, degs):
    return pl.pallas_call(
        _k1b_body,
        grid=(NP // _R,),
        in_specs=[
            pl.BlockSpec((_R, D_HID), lambda i: (i, 0)),
            pl.BlockSpec((NC, _R), lambda i: (0, i)),
        ],
        out_specs=[
            pl.BlockSpec((_R, D_HID), lambda i: (i, 0)),
            pl.BlockSpec((_R, 1), lambda i: (i, 0)),
        ],
        out_shape=[
            jax.ShapeDtypeStruct((NP, D_HID), jnp.float32),
            jax.ShapeDtypeStruct((NP, 1), jnp.float32),
        ],
    )(h1, degs)


def _tc_mid(acc1, g1, dinv, b1r, W2):
    return pl.pallas_call(
        _k3_body,
        grid=(NP // _R,),
        in_specs=[
            pl.BlockSpec((NC, _R, D_HID), lambda i: (0, i, 0)),
            pl.BlockSpec((_R, D_HID), lambda i: (i, 0)),
            pl.BlockSpec((_R, 1), lambda i: (i, 0)),
            pl.BlockSpec((1, D_HID), lambda i: (0, 0)),
            pl.BlockSpec((D_HID, D_LAT), lambda i: (0, 0)),
        ],
        out_specs=pl.BlockSpec((_R, D_LAT), lambda i: (i, 0)),
        out_shape=jax.ShapeDtypeStruct((NP, D_LAT), jnp.float32),
    )(acc1, g1, dinv, b1r, W2)


_R5 = 1000


def _tc_final(acc2, g2, dinv, b2r):
    return pl.pallas_call(
        _k5_body,
        grid=(N // _R5,),
        in_specs=[
            pl.BlockSpec((NC, _R5, D_LAT), lambda i: (0, i, 0)),
            pl.BlockSpec((_R5, D_LAT), lambda i: (i, 0)),
            pl.BlockSpec((_R5, 1), lambda i: (i, 0)),
            pl.BlockSpec((1, D_LAT), lambda i: (0, 0)),
        ],
        out_specs=pl.BlockSpec((_R5, D_LAT), lambda i: (i, 0)),
        out_shape=jax.ShapeDtypeStruct((N, D_LAT), jnp.float32),
    )(acc2, g2, dinv, b2r)


def kernel(x, edge_index, W1, b1, W2, b2):
    ei = edge_index.astype(jnp.int32)
    epw = E_PAD // (NC * NS)
    pad_idx = N + (jnp.arange(E_PAD - E, dtype=jnp.int32) % (NP - N))
    tail = jnp.concatenate(
        [ei[:, (NC * NS - 1) * epw:], jnp.stack([pad_idx, pad_idx])], axis=1
    )
    ei3a = ei.reshape(2, E // 32, 32)
    ei3b = ei.reshape(2, E // 128, 128)
    tail3a = tail.reshape(2, epw // 32, 32)
    tail3b = tail.reshape(2, epw // 128, 128)

    h1 = _tc_matmul(x, W1)
    degs = _deg_kernel(ei3b, tail3b)
    g1, dinv = _tc_scale(h1, degs)
    acc1 = _agg_hid(g1, ei3a, tail3a)
    g2 = _tc_mid(acc1, g1, dinv, b1.reshape(1, D_HID), W2)
    acc2 = _agg_lat(g2, ei3b, tail3b)
    return _tc_final(acc2, g2, dinv, b2.reshape(1, D_LAT))
